# baseline (speedup 1.0000x reference)
import functools

import jax
import jax.numpy as jnp
from jax import lax
from jax.experimental import pallas as pl
from jax.experimental.pallas import tpu as pltpu
from jax.experimental.pallas import tpu_sc as plsc

_NC, _NS, _L = 2, 16, 16
_NW = _NC * _NS

_B, _T, _H, _W = 16, 128, 64, 64
_DIM_S = 256
_PIX = 128
_HH = _H * _W // _PIX
_GROUPS = _B * _HH
_GPW = _GROUPS // _NW
_NSUB = _PIX // _L


@functools.partial(
    pl.kernel,
    out_type=jax.ShapeDtypeStruct((_B, _DIM_S, _HH, _PIX), jnp.int32),
    mesh=plsc.VectorSubcoreMesh(core_axis_name="c", subcore_axis_name="s"),
    scratch_types=[
        pltpu.VMEM((2, _T, _PIX), jnp.int32),
        pltpu.VMEM((2, _DIM_S, _PIX), jnp.int32),
        pltpu.SemaphoreType.DMA,
        pltpu.SemaphoreType.DMA,
        pltpu.SemaphoreType.DMA,
        pltpu.SemaphoreType.DMA,
    ],
    compiler_params=pltpu.CompilerParams(
        needs_layout_passes=False, use_tc_tiling_on_sc=True),
)
def _spike_hist(in_hbm, out_hbm, inbuf, hist, si0, si1, so0, so1):
    wid = lax.axis_index("s") * _NC + lax.axis_index("c")
    lanes = lax.iota(jnp.int32, _L)
    ones = jnp.ones((_L,), jnp.int32)
    zeros = jnp.zeros((_L,), jnp.int32)
    sin = (si0, si1)
    sout = (so0, so1)

    def bh(g):
        gid = g * _NW + wid
        return gid // _HH, gid % _HH

    def start_in(g, slot, sem):
        b, hh = bh(g)
        return pltpu.async_copy(in_hbm.at[b, :, hh, :], inbuf.at[slot], sem)

    def start_out(g, slot, sem):
        b, hh = bh(g)
        return pltpu.async_copy(hist.at[slot], out_hbm.at[b, :, hh, :], sem)

    start_in(0, 0, sin[0])
    start_in(1, 1, sin[1])

    def pair_body(g2, carry):
        for slot in range(2):
            g = g2 * 2 + slot
            b, hh = bh(g)

            @pl.when(g2 > 0)
            def _drain():
                pltpu.make_async_copy(
                    hist.at[slot], out_hbm.at[b, :, hh, :], sout[slot]).wait()

            @plsc.parallel_loop(0, _DIM_S, unroll=4)
            def _zero(i):
                for k in range(_NSUB):
                    hist[slot, i, pl.ds(k * _L, _L)] = zeros

            pltpu.make_async_copy(
                in_hbm.at[b, :, hh, :], inbuf.at[slot], sin[slot]).wait()

            hs = hist.at[slot]

            @plsc.parallel_loop(0, _T, unroll=8)
            def _t_body(t):
                for k in range(_NSUB):
                    vals = inbuf[slot, t, pl.ds(k * _L, _L)]
                    plsc.addupdate_scatter(hs, [vals, lanes + k * _L], ones)

            @pl.when(g + 2 < _GPW)
            def _prefetch():
                start_in(g + 2, slot, sin[slot])

            start_out(g, slot, sout[slot])
        return carry

    lax.fori_loop(0, _GPW // 2, pair_body, 0)

    for slot in range(2):
        g = _GPW - 2 + slot
        b, hh = bh(g)
        pltpu.make_async_copy(
            hist.at[slot], out_hbm.at[b, :, hh, :], sout[slot]).wait()


def kernel(input, dim_s):
    del dim_s
    out = _spike_hist(input.reshape(_B, _T, _HH, _PIX))
    return out.reshape(_B, _DIM_S, _H, _W)


# speedup vs baseline: 1.1047x; 1.1047x over previous
import functools

import jax
import jax.numpy as jnp
from jax import lax
from jax.experimental import pallas as pl
from jax.experimental.pallas import tpu as pltpu
from jax.experimental.pallas import tpu_sc as plsc

_NC, _NS, _L = 2, 16, 16
_NW = _NC * _NS

_B, _T, _H, _W = 16, 128, 64, 64
_DIM_S = 256
_PIX = 128
_HH = _H * _W // _PIX
_GROUPS = _B * _HH
_GPW = _GROUPS // _NW
_NSUB = _PIX // _L


@functools.partial(
    pl.kernel,
    out_type=jax.ShapeDtypeStruct((_B, _DIM_S, _HH, _PIX), jnp.int32),
    mesh=plsc.VectorSubcoreMesh(core_axis_name="c", subcore_axis_name="s"),
    scratch_types=[
        pltpu.VMEM((2, _T, _PIX), jnp.int32),
        pltpu.VMEM((2, _DIM_S, _PIX), jnp.int32),
        pltpu.SemaphoreType.DMA,
        pltpu.SemaphoreType.DMA,
        pltpu.SemaphoreType.DMA,
        pltpu.SemaphoreType.DMA,
    ],
    compiler_params=pltpu.CompilerParams(
        needs_layout_passes=False, use_tc_tiling_on_sc=False),
)
def _spike_hist(in_hbm, out_hbm, inbuf, hist, si0, si1, so0, so1):
    wid = lax.axis_index("s") * _NC + lax.axis_index("c")
    lanes = lax.iota(jnp.int32, _L)
    ones = jnp.ones((_L,), jnp.int32)
    zeros = jnp.zeros((_L,), jnp.int32)
    sin = (si0, si1)
    sout = (so0, so1)

    def bh(g):
        gid = g * _NW + wid
        return gid // _HH, gid % _HH

    def start_in(g, slot, sem):
        b, hh = bh(g)
        return pltpu.async_copy(in_hbm.at[b, :, hh, :], inbuf.at[slot], sem)

    def start_out(g, slot, sem):
        b, hh = bh(g)
        return pltpu.async_copy(hist.at[slot], out_hbm.at[b, :, hh, :], sem)

    start_in(0, 0, sin[0])
    start_in(1, 1, sin[1])

    def pair_body(g2, carry):
        for slot in range(2):
            g = g2 * 2 + slot
            b, hh = bh(g)

            @pl.when(g2 > 0)
            def _drain():
                pltpu.make_async_copy(
                    hist.at[slot], out_hbm.at[b, :, hh, :], sout[slot]).wait()

            @plsc.parallel_loop(0, _DIM_S, unroll=4)
            def _zero(i):
                for k in range(_NSUB):
                    hist[slot, i, pl.ds(k * _L, _L)] = zeros

            pltpu.make_async_copy(
                in_hbm.at[b, :, hh, :], inbuf.at[slot], sin[slot]).wait()

            hs = hist.at[slot]

            @plsc.parallel_loop(0, _T, unroll=8)
            def _t_body(t):
                for k in range(_NSUB):
                    vals = inbuf[slot, t, pl.ds(k * _L, _L)]
                    plsc.addupdate_scatter(hs, [vals, lanes + k * _L], ones)

            @pl.when(g + 2 < _GPW)
            def _prefetch():
                start_in(g + 2, slot, sin[slot])

            start_out(g, slot, sout[slot])
        return carry

    lax.fori_loop(0, _GPW // 2, pair_body, 0)

    for slot in range(2):
        g = _GPW - 2 + slot
        b, hh = bh(g)
        pltpu.make_async_copy(
            hist.at[slot], out_hbm.at[b, :, hh, :], sout[slot]).wait()


def kernel(input, dim_s):
    del dim_s
    out = _spike_hist(input.reshape(_B, _T, _HH, _PIX))
    return out.reshape(_B, _DIM_S, _H, _W)
